# baseline (speedup 1.0000x reference)
import functools
import math

import jax
import jax.numpy as jnp
from jax import lax
from jax.experimental import pallas as pl
from jax.experimental.pallas import tpu as pltpu
from jax.experimental.pallas import tpu_sc as plsc

D_MODEL = 64
SCALE = math.sqrt(D_MODEL)

NUM_CORES = 2
NUM_SUBCORES = 16
NUM_WORKERS = NUM_CORES * NUM_SUBCORES
LANES = 16

BLK = 128
NBUF = 4


@functools.lru_cache(maxsize=None)
def _build_transpose(vocab: int):
    num_jobs = vocab // BLK
    last_full = num_jobs - 1
    jobs_per_w = (num_jobs + NUM_WORKERS - 1) // NUM_WORKERS
    half = (jobs_per_w + 2) // 2
    mesh = plsc.VectorSubcoreMesh(
        core_axis_name="c", subcore_axis_name="s",
        num_cores=NUM_CORES, num_subcores=NUM_SUBCORES,
    )

    @functools.partial(
        pl.kernel,
        out_type=jax.ShapeDtypeStruct((vocab, 2 * D_MODEL), jnp.float32),
        mesh=mesh,
        scratch_types=[
            pltpu.VMEM((2, D_MODEL, BLK), jnp.float32),
            pltpu.VMEM((2, BLK, 2 * D_MODEL), jnp.float32),
        ] + [pltpu.SemaphoreType.DMA] * 4,
        compiler_params=pltpu.CompilerParams(
            use_tc_tiling_on_sc=True, needs_layout_passes=False),
    )
    def transpose_kernel(wt_hbm, tail_hbm, out_hbm, tbuf, obuf, *sems):
        isems = sems[:2]
        osems = sems[2:]
        wid = lax.axis_index("s") * NUM_CORES + lax.axis_index("c")

        def job_id(t):
            return jnp.minimum(t * NUM_WORKERS + wid, last_full)

        def start_in(t, b):
            pltpu.async_copy(
                wt_hbm.at[:, pl.ds(job_id(t) * BLK, BLK)],
                tbuf.at[b], isems[b])

        start_in(0, 0)

        def outer(t2, carry):
            for b in range(2):
                t = t2 * 2 + b
                pltpu.make_async_copy(
                    wt_hbm.at[:, pl.ds(0, BLK)], tbuf.at[b], isems[b]).wait()
                start_in(t + 1, 1 - b)

                @pl.when(t2 > 0)
                def _wait_out(b=b):
                    pltpu.make_async_copy(
                        obuf.at[b], out_hbm.at[pl.ds(0, BLK)],
                        osems[b]).wait()

                def row_body(r, c, b=b):
                    rv = lax.broadcast(r, (LANES,))
                    ob = obuf.at[b]
                    for j in range(D_MODEL // LANES):
                        dv = lax.iota(jnp.int32, LANES) + (j * LANES)
                        v = plsc.load_gather(tbuf.at[b], [dv, rv])
                        ob[r, pl.ds(j * LANES, LANES)] = v * SCALE
                    return c

                lax.fori_loop(0, BLK, row_body, 0)
                pltpu.async_copy(
                    obuf.at[b], out_hbm.at[pl.ds(job_id(t) * BLK, BLK)],
                    osems[b])
            return carry

        lax.fori_loop(0, half, outer, 0)

        pltpu.make_async_copy(
            wt_hbm.at[:, pl.ds(0, BLK)], tbuf.at[0], isems[0]).wait()
        for b in range(2):
            pltpu.make_async_copy(
                obuf.at[b], out_hbm.at[pl.ds(0, BLK)], osems[b]).wait()

        tail = vocab - num_jobs * BLK
        @pl.when(wid == 0)
        def _tail():
            pltpu.sync_copy(tail_hbm, obuf.at[0, pl.ds(0, tail)])

            def row_body(r, c):
                ob = obuf.at[0]
                for j in range(D_MODEL // LANES):
                    sl = pl.ds(j * LANES, LANES)
                    ob[r, sl] = ob[r, sl] * SCALE
                return c

            lax.fori_loop(0, tail, row_body, 0)
            pltpu.sync_copy(
                obuf.at[0, pl.ds(0, tail)],
                out_hbm.at[pl.ds(num_jobs * BLK, tail)])

    return transpose_kernel


@functools.lru_cache(maxsize=None)
def _build_gather(num_blocks: int, vocab: int):
    blocks_per_w = num_blocks // NUM_WORKERS
    outer_iters = blocks_per_w // NBUF
    mesh = plsc.VectorSubcoreMesh(
        core_axis_name="c", subcore_axis_name="s",
        num_cores=NUM_CORES, num_subcores=NUM_SUBCORES,
    )

    @functools.partial(
        pl.kernel,
        out_type=jax.ShapeDtypeStruct((num_blocks * BLK, 2 * D_MODEL),
                                      jnp.float32),
        mesh=mesh,
        scratch_types=[
            pltpu.VMEM((blocks_per_w, BLK), jnp.int32),
            pltpu.VMEM((NBUF, BLK, 2 * D_MODEL), jnp.float32),
        ] + [pltpu.SemaphoreType.DMA] * (2 * NBUF),
        compiler_params=pltpu.CompilerParams(use_tc_tiling_on_sc=True),
    )
    def gather_kernel(idx_hbm, table_hbm, out_hbm, idx_v, ibuf, *sems):
        gsems = sems[:NBUF]
        ssems = sems[NBUF:]
        wid = lax.axis_index("s") * NUM_CORES + lax.axis_index("c")
        blk0 = wid * blocks_per_w
        pltpu.sync_copy(idx_hbm.at[pl.ds(blk0, blocks_per_w)], idx_v)

        for b in range(NBUF):
            pltpu.async_copy(
                table_hbm.at[idx_v.at[b]], ibuf.at[b], gsems[b])

        def outer(t, carry):
            for b in range(NBUF):
                g = t * NBUF + b
                pltpu.make_async_copy(
                    table_hbm.at[idx_v.at[0]], ibuf.at[b], gsems[b]).wait()
                pltpu.async_copy(
                    ibuf.at[b], out_hbm.at[pl.ds((blk0 + g) * BLK, BLK)],
                    ssems[b])

                @pl.when(t < outer_iters - 1)
                def _next(b=b, g=g):
                    pltpu.make_async_copy(
                        ibuf.at[b], out_hbm.at[pl.ds(0, BLK)],
                        ssems[b]).wait()
                    pltpu.async_copy(
                        table_hbm.at[idx_v.at[g + NBUF]], ibuf.at[b],
                        gsems[b])
            return carry

        lax.fori_loop(0, outer_iters, outer, 0)

        for b in range(NBUF):
            pltpu.make_async_copy(
                ibuf.at[b], out_hbm.at[pl.ds(0, BLK)], ssems[b]).wait()

    return gather_kernel


def kernel(x, embed_weight):
    b, s = x.shape
    n = b * s
    vocab = embed_weight.shape[0]
    idx = x.reshape(n // BLK, BLK).astype(jnp.int32)
    tail = vocab - (vocab // BLK) * BLK
    tail_pad = jnp.pad(
        embed_weight[vocab - tail:], ((0, 0), (0, 2 * D_MODEL - D_MODEL)))
    table = _build_transpose(vocab)(embed_weight.T, tail_pad)
    out = _build_gather(n // BLK, vocab)(idx, table)
    return out[:, :D_MODEL].reshape(b, s, D_MODEL)


# speedup vs baseline: 1.4576x; 1.4576x over previous
import functools
import math

import jax
import jax.numpy as jnp
from jax import lax
from jax.experimental import pallas as pl
from jax.experimental.pallas import tpu as pltpu
from jax.experimental.pallas import tpu_sc as plsc

D_MODEL = 64
SCALE = math.sqrt(D_MODEL)

NUM_CORES = 2
NUM_SUBCORES = 16
NUM_WORKERS = NUM_CORES * NUM_SUBCORES
LANES = 16

BLK = 128
NBUF = 8
ROWS_PER_ITER = 8


@functools.lru_cache(maxsize=None)
def _build_transpose(vocab: int):
    num_jobs = vocab // BLK
    last_full = num_jobs - 1
    jobs_per_w = (num_jobs + NUM_WORKERS - 1) // NUM_WORKERS
    half = (jobs_per_w + 2) // 2
    mesh = plsc.VectorSubcoreMesh(
        core_axis_name="c", subcore_axis_name="s",
        num_cores=NUM_CORES, num_subcores=NUM_SUBCORES,
    )

    @functools.partial(
        pl.kernel,
        out_type=jax.ShapeDtypeStruct((vocab // 2, 2 * D_MODEL), jnp.float32),
        mesh=mesh,
        scratch_types=[
            pltpu.VMEM((2, D_MODEL, BLK), jnp.float32),
            pltpu.VMEM((2, BLK // 2, 2 * D_MODEL), jnp.float32),
        ] + [pltpu.SemaphoreType.DMA] * 4,
        compiler_params=pltpu.CompilerParams(
            use_tc_tiling_on_sc=True, needs_layout_passes=False),
    )
    def transpose_kernel(wt_hbm, tail_hbm, out_hbm, tbuf, obuf, *sems):
        isems = sems[:2]
        osems = sems[2:]
        wid = lax.axis_index("s") * NUM_CORES + lax.axis_index("c")

        def job_id(t):
            return jnp.minimum(t * NUM_WORKERS + wid, last_full)

        def start_in(t, b):
            pltpu.async_copy(
                wt_hbm.at[:, pl.ds(job_id(t) * BLK, BLK)],
                tbuf.at[b], isems[b])

        dvs = [lax.iota(jnp.int32, LANES) + (j * LANES)
               for j in range(D_MODEL // LANES)]

        start_in(0, 0)

        def outer(t2, carry):
            for b in range(2):
                t = t2 * 2 + b
                pltpu.make_async_copy(
                    wt_hbm.at[:, pl.ds(0, BLK)], tbuf.at[b], isems[b]).wait()
                start_in(t + 1, 1 - b)

                @pl.when(t2 > 0)
                def _wait_out(b=b):
                    pltpu.make_async_copy(
                        obuf.at[b], out_hbm.at[pl.ds(0, BLK // 2)],
                        osems[b]).wait()

                def rows_body(ri, c, b=b):
                    r0 = ri * ROWS_PER_ITER
                    tb = tbuf.at[b]
                    ob = obuf.at[b]
                    vs = []
                    for k in range(ROWS_PER_ITER):
                        rv = lax.broadcast(r0 + k, (LANES,))
                        for j in range(D_MODEL // LANES):
                            vs.append(plsc.load_gather(tb, [dvs[j], rv]))
                    i = 0
                    for k in range(ROWS_PER_ITER):
                        for j in range(D_MODEL // LANES):
                            dst = pl.ds((k % 2) * D_MODEL + j * LANES, LANES)
                            p = ri * (ROWS_PER_ITER // 2) + (k // 2)
                            ob[p, dst] = vs[i] * SCALE
                            i += 1
                    return c

                lax.fori_loop(0, BLK // ROWS_PER_ITER, rows_body, 0)
                pltpu.async_copy(
                    obuf.at[b],
                    out_hbm.at[pl.ds(job_id(t) * (BLK // 2), BLK // 2)],
                    osems[b])
            return carry

        lax.fori_loop(0, half, outer, 0)

        pltpu.make_async_copy(
            wt_hbm.at[:, pl.ds(0, BLK)], tbuf.at[0], isems[0]).wait()
        for b in range(2):
            pltpu.make_async_copy(
                obuf.at[b], out_hbm.at[pl.ds(0, BLK // 2)], osems[b]).wait()

        tail = vocab - num_jobs * BLK
        @pl.when(wid == 0)
        def _tail():
            pltpu.sync_copy(tail_hbm, tbuf.at[0])

            def row_body(q, c):
                tb = tbuf.at[0]
                ob = obuf.at[0]
                for h in range(2):
                    for j in range(D_MODEL // LANES):
                        src = pl.ds(j * LANES, LANES)
                        dst = pl.ds(h * D_MODEL + j * LANES, LANES)
                        ob[q, dst] = tb[2 * q + h, src] * SCALE
                return c

            lax.fori_loop(0, tail // 2, row_body, 0)
            pltpu.sync_copy(
                obuf.at[0, pl.ds(0, tail // 2)],
                out_hbm.at[pl.ds(num_jobs * (BLK // 2), tail // 2)])

    return transpose_kernel


@functools.lru_cache(maxsize=None)
def _build_gather(num_blocks: int, vocab: int):
    blocks_per_w = num_blocks // NUM_WORKERS
    outer_iters = blocks_per_w // NBUF
    mesh = plsc.VectorSubcoreMesh(
        core_axis_name="c", subcore_axis_name="s",
        num_cores=NUM_CORES, num_subcores=NUM_SUBCORES,
    )

    @functools.partial(
        pl.kernel,
        out_type=jax.ShapeDtypeStruct((num_blocks * BLK, 2 * D_MODEL),
                                      jnp.float32),
        mesh=mesh,
        scratch_types=[
            pltpu.VMEM((blocks_per_w, BLK), jnp.int32),
            pltpu.VMEM((NBUF, BLK, D_MODEL), jnp.float32),
        ] + [pltpu.SemaphoreType.DMA] * (2 * NBUF),
        compiler_params=pltpu.CompilerParams(use_tc_tiling_on_sc=False),
    )
    def gather_kernel(idx_hbm, table_hbm, out_hbm, idx_v, ibuf, *sems):
        gsems = sems[:NBUF]
        ssems = sems[NBUF:]
        wid = lax.axis_index("s") * NUM_CORES + lax.axis_index("c")
        blk0 = wid * blocks_per_w
        pltpu.sync_copy(idx_hbm.at[pl.ds(blk0, blocks_per_w)], idx_v)

        for b in range(NBUF):
            pltpu.async_copy(
                table_hbm.at[idx_v.at[b]], ibuf.at[b], gsems[b])

        def outer(t, carry):
            for b in range(NBUF):
                g = t * NBUF + b
                pltpu.make_async_copy(
                    table_hbm.at[idx_v.at[0]], ibuf.at[b], gsems[b]).wait()
                pltpu.async_copy(
                    ibuf.at[b],
                    out_hbm.at[pl.ds((blk0 + g) * BLK, BLK),
                               pl.ds(0, D_MODEL)],
                    ssems[b])

                @pl.when(t < outer_iters - 1)
                def _next(b=b, g=g):
                    pltpu.make_async_copy(
                        ibuf.at[b],
                        out_hbm.at[pl.ds(0, BLK), pl.ds(0, D_MODEL)],
                        ssems[b]).wait()
                    pltpu.async_copy(
                        table_hbm.at[idx_v.at[g + NBUF]], ibuf.at[b],
                        gsems[b])
            return carry

        lax.fori_loop(0, outer_iters, outer, 0)

        for b in range(NBUF):
            pltpu.make_async_copy(
                ibuf.at[b],
                out_hbm.at[pl.ds(0, BLK), pl.ds(0, D_MODEL)],
                ssems[b]).wait()

    return gather_kernel


def kernel(x, embed_weight):
    b, s = x.shape
    n = b * s
    vocab = embed_weight.shape[0]
    idx = x.reshape(n // BLK, BLK).astype(jnp.int32)
    tail = vocab - (vocab // BLK) * BLK
    tail_pad = jnp.pad(
        embed_weight[vocab - tail:], ((0, 0), (0, 2 * D_MODEL - D_MODEL)))
    table2 = _build_transpose(vocab)(embed_weight.T, tail_pad)
    table = table2.reshape(vocab, D_MODEL)
    out = _build_gather(n // BLK, vocab)(idx, table)
    return out[:, :D_MODEL].reshape(b, s, D_MODEL)
